# baseline (speedup 1.0000x reference)
import functools

import jax
import jax.numpy as jnp
from jax import lax
from jax.experimental import pallas as pl
from jax.experimental.pallas import tpu as pltpu
from jax.experimental.pallas import tpu_sc as plsc

B = 4
N = 4096
D = 768
NCEN = 256
HALF = NCEN // 2

_HIGH = jax.lax.Precision.DEFAULT


def _fps_steps(cx, cy, cz, ii, bb, lpx, lpy, lpz, dist, t0, t1, base,
               flat_ref, ccx_ref, ccy_ref, ccz_ref):
    for t in range(t0, t1):
        dx = cx - lpx
        dy = cy - lpy
        dz = cz - lpz
        d = dx * dx + dy * dy + dz * dz
        dist = d if dist is None else jnp.minimum(dist, d)
        m = jnp.max(dist, axis=1, keepdims=True)
        cand = jnp.where(dist == m, ii, jnp.int32(2 ** 30))
        nxt = jnp.min(cand, axis=1, keepdims=True)
        msk = ii == nxt
        lpx = jnp.sum(jnp.where(msk, cx, 0.0), axis=1, keepdims=True)
        lpy = jnp.sum(jnp.where(msk, cy, 0.0), axis=1, keepdims=True)
        lpz = jnp.sum(jnp.where(msk, cz, 0.0), axis=1, keepdims=True)
        s = t - base
        flat_ref[:, s:s + 1, :] = (nxt + bb)[:, :, None]
        ccx_ref[:, s:s + 1, :] = lpx[:, :, None]
        ccy_ref[:, s:s + 1, :] = lpy[:, :, None]
        ccz_ref[:, s:s + 1, :] = lpz[:, :, None]
    return lpx, lpy, lpz, dist


def _fps1_body(cx_ref, cy_ref, cz_ref,
               flat_ref, ccx_ref, ccy_ref, ccz_ref,
               dist_ref, lp_ref):
    cx = cx_ref[...]
    cy = cy_ref[...]
    cz = cz_ref[...]
    ii = lax.broadcasted_iota(jnp.int32, (B, N), 1)
    bb = lax.broadcasted_iota(jnp.int32, (B, 1), 0) * N
    lpx = cx[:, 0:1]
    lpy = cy[:, 0:1]
    lpz = cz[:, 0:1]
    flat_ref[:, 0:1, :] = bb[:, :, None]
    ccx_ref[:, 0:1, :] = lpx[:, :, None]
    ccy_ref[:, 0:1, :] = lpy[:, :, None]
    ccz_ref[:, 0:1, :] = lpz[:, :, None]
    lpx, lpy, lpz, dist = _fps_steps(cx, cy, cz, ii, bb, lpx, lpy, lpz,
                                     None, 1, HALF, 0,
                                     flat_ref, ccx_ref, ccy_ref, ccz_ref)
    dist_ref[...] = dist
    lp_ref[...] = jnp.concatenate([lpx, lpy, lpz], axis=1)


def _fps2_body(cx_ref, cy_ref, cz_ref, dist_ref, lp_ref,
               flat_ref, ccx_ref, ccy_ref, ccz_ref):
    cx = cx_ref[...]
    cy = cy_ref[...]
    cz = cz_ref[...]
    ii = lax.broadcasted_iota(jnp.int32, (B, N), 1)
    bb = lax.broadcasted_iota(jnp.int32, (B, 1), 0) * N
    lpx = lp_ref[:, 0:1]
    lpy = lp_ref[:, 1:2]
    lpz = lp_ref[:, 2:3]
    _fps_steps(cx, cy, cz, ii, bb, lpx, lpy, lpz, dist_ref[...],
               HALF, NCEN, HALF, flat_ref, ccx_ref, ccy_ref, ccz_ref)


def _fps1(cx, cy, cz):
    return pl.pallas_call(
        _fps1_body,
        out_shape=[
            jax.ShapeDtypeStruct((B, HALF, 1), jnp.int32),
            jax.ShapeDtypeStruct((B, HALF, 1), jnp.float32),
            jax.ShapeDtypeStruct((B, HALF, 1), jnp.float32),
            jax.ShapeDtypeStruct((B, HALF, 1), jnp.float32),
            jax.ShapeDtypeStruct((B, N), jnp.float32),
            jax.ShapeDtypeStruct((B, 3), jnp.float32),
        ],
    )(cx, cy, cz)


def _fps2(cx, cy, cz, dist, lp):
    return pl.pallas_call(
        _fps2_body,
        out_shape=[
            jax.ShapeDtypeStruct((B, HALF, 1), jnp.int32),
            jax.ShapeDtypeStruct((B, HALF, 1), jnp.float32),
            jax.ShapeDtypeStruct((B, HALF, 1), jnp.float32),
            jax.ShapeDtypeStruct((B, HALF, 1), jnp.float32),
        ],
    )(cx, cy, cz, dist, lp)


_NW = 32
_HROWS = B * HALF
_RPW = _HROWS // _NW


def _sc_gather(table, idx):
    mesh = plsc.VectorSubcoreMesh(core_axis_name="c", subcore_axis_name="s")

    @functools.partial(
        pl.kernel,
        mesh=mesh,
        out_type=jax.ShapeDtypeStruct((_HROWS, D), jnp.float32),
        scratch_types=[
            pltpu.VMEM((_RPW,), jnp.int32),
            pltpu.VMEM((_RPW, D), jnp.float32),
            pltpu.SemaphoreType.DMA,
        ],
    )
    def gather_kernel(table_hbm, idx_hbm, out_hbm, idx_v, rows_v, sem):
        wid = lax.axis_index("s") * 2 + lax.axis_index("c")
        base = wid * _RPW
        pltpu.sync_copy(idx_hbm.at[pl.ds(base, _RPW)], idx_v)
        pltpu.async_copy(table_hbm.at[idx_v], rows_v, sem).wait()
        pltpu.sync_copy(rows_v, out_hbm.at[pl.ds(base, _RPW)])

    return gather_kernel(table, idx)


def _dot(a, b):
    return jax.lax.dot_general(a, b, (((1,), (0,)), ((), ())),
                               precision=_HIGH,
                               preferred_element_type=jnp.float32)


def _sigmoid(x):
    return 1.0 / (1.0 + jnp.exp(-x))


def _select_body(ce1_ref, ce2_ref, ccxc_ref, ccyc_ref, cczc_ref, ccz2c_ref,
                 ccxr_ref, ccyr_ref, cczr_ref,
                 wg1_ref, bg1_ref, wg2_ref, bg2_ref,
                 wc1_ref, bc1_ref, wc2_ref, bc2_ref,
                 wd1_ref, bd1_ref, wd2_ref, bd2_ref,
                 wp1_ref, bp1_ref, wp2_ref, bp2_ref,
                 lng_ref, lnb_ref, out_ref, sel_ref, s_ref):
    ce1 = ce1_ref[...]
    ce2 = ce2_ref[...]
    ce = jnp.concatenate(
        [blk for b in range(B)
         for blk in (ce1[b * HALF:(b + 1) * HALF, :],
                     ce2[b * HALF:(b + 1) * HALF, :])], axis=0)

    hg = jnp.maximum(_dot(ce, wg1_ref[...]) + bg1_ref[...], 0.0)
    pg = _sigmoid(_dot(hg, wg2_ref[...]) + bg2_ref[...])
    hc = jnp.maximum(_dot(ce, wc1_ref[...]) + bc1_ref[...], 0.0)
    pc = _sigmoid(_dot(hc, wc2_ref[...]) + bc2_ref[...])
    hd = jnp.maximum(_dot(ce, wd1_ref[...]) + bd1_ref[...], 0.0)
    pd = _sigmoid(_dot(hd, wd2_ref[...]) + bd2_ref[...])

    def topk_onehot(row, k, srow):
        w = row.shape[1]
        iot = lax.broadcasted_iota(jnp.int32, (1, w), 1)
        big = jnp.int32(2 ** 30)
        for j in range(k):
            m = jnp.max(row, axis=1, keepdims=True)
            idxv = jnp.min(jnp.where(row == m, iot, big), axis=1,
                           keepdims=True)
            oh = iot == idxv
            s_ref[srow + j:srow + j + 1, 0:w] = jnp.where(oh, 1.0, 0.0)
            row = jnp.where(oh, -jnp.inf, row)

    for b in range(B):
        s_ref[...] = jnp.zeros((40, NCEN), jnp.float32)
        z = jnp.concatenate([cczc_ref[b], ccz2c_ref[b]], axis=0)
        hr = _sigmoid((z - jnp.mean(z)) / 5.0)
        sg = 1.0 + hr * 0.95
        pg_b = pg[b * NCEN:(b + 1) * NCEN, :]
        score_g = jnp.mean(pg_b * sg, axis=1, keepdims=True)
        topk_onehot(jnp.reshape(score_g, (1, NCEN)), 16, 0)

        pc_b = pc[b * NCEN:b * NCEN + 128, :]
        score_c = jnp.mean(pc_b, axis=1, keepdims=True)
        topk_onehot(jnp.reshape(score_c, (1, 128)), 16, 16)

        pd_b = pd[b * NCEN:b * NCEN + 64, :]
        xi = ccxc_ref[b, 0:64, :]
        yi = ccyc_ref[b, 0:64, :]
        zi = cczc_ref[b, 0:64, :]
        xj = ccxr_ref[b:b + 1, 0:64]
        yj = ccyr_ref[b:b + 1, 0:64]
        zj = cczr_ref[b:b + 1, 0:64]
        dxx = xi - xj
        dyy = yi - yj
        dzz = zi - zj
        d2 = dxx * dxx + dyy * dyy + dzz * dzz
        dens = jnp.sum(jnp.where(d2 < 0.25, 1.0, 0.0), axis=1, keepdims=True)
        sd = 1.0 + dens / 64.0 * 0.95
        score_d = jnp.mean(pd_b * sd, axis=1, keepdims=True)
        topk_onehot(jnp.reshape(score_d, (1, 64)), 8, 32)

        ce_b = ce[b * NCEN:(b + 1) * NCEN, :]
        sel_ref[b * 40:(b + 1) * 40, :] = _dot(s_ref[...], ce_b)

    sel = sel_ref[...]
    h2 = jnp.maximum(_dot(sel, wp1_ref[...]) + bp1_ref[...], 0.0)
    o = _dot(h2, wp2_ref[...]) + bp2_ref[...]
    mu = jnp.mean(o, axis=1, keepdims=True)
    var = jnp.mean((o - mu) * (o - mu), axis=1, keepdims=True)
    res = (o - mu) / jnp.sqrt(var + 1e-5) * lng_ref[...] + lnb_ref[...]
    out_ref[...] = res.reshape(B, 40, D)


def _select(ce1, ce2, ccxc, ccyc, cczc, ccz2c, ccxr, ccyr, cczr, *weights):
    return pl.pallas_call(
        _select_body,
        out_shape=jax.ShapeDtypeStruct((B, 40, D), jnp.float32),
        scratch_shapes=[pltpu.VMEM((B * 40, D), jnp.float32),
                        pltpu.VMEM((40, NCEN), jnp.float32)],
    )(ce1, ce2, ccxc, ccyc, cczc, ccz2c, ccxr, ccyr, cczr, *weights)


def kernel(point_features, point_coords, Wg1, bg1, Wg2, bg2, Wc1, bc1, Wc2,
           bc2, Wd1, bd1, Wd2, bd2, Wp1, bp1, Wp2, bp2, ln_g, ln_b):
    cx = point_coords[:, :, 0]
    cy = point_coords[:, :, 1]
    cz = point_coords[:, :, 2]
    table = point_features.reshape(B * N, D)

    flat1, ccx1, ccy1, ccz1, dist, lp = _fps1(cx, cy, cz)
    ce1 = _sc_gather(table, flat1.reshape(_HROWS))
    flat2, ccx2, ccy2, ccz2 = _fps2(cx, cy, cz, dist, lp)
    ce2 = _sc_gather(table, flat2.reshape(_HROWS))

    ccxr = ccx1.reshape(B, HALF)
    ccyr = ccy1.reshape(B, HALF)
    cczr = ccz1.reshape(B, HALF)

    out = _select(
        ce1, ce2, ccx1, ccy1, ccz1, ccz2, ccxr, ccyr, cczr,
        Wg1, bg1.reshape(1, -1), Wg2, bg2.reshape(1, -1),
        Wc1, bc1.reshape(1, -1), Wc2, bc2.reshape(1, -1),
        Wd1, bd1.reshape(1, -1), Wd2, bd2.reshape(1, -1),
        Wp1, bp1.reshape(1, -1), Wp2, bp2.reshape(1, -1),
        ln_g.reshape(1, -1), ln_b.reshape(1, -1),
    )
    return out


# speedup vs baseline: 1.0237x; 1.0237x over previous
import functools

import jax
import jax.numpy as jnp
from jax import lax
from jax.experimental import pallas as pl
from jax.experimental.pallas import tpu as pltpu
from jax.experimental.pallas import tpu_sc as plsc

B = 4
N = 4096
D = 768
NCEN = 256

_HIGH = jax.lax.Precision.DEFAULT


def _fps_body(cx_ref, cy_ref, cz_ref, flat_ref, ccx_ref, ccy_ref, ccz_ref):
    cx = cx_ref[...]
    cy = cy_ref[...]
    cz = cz_ref[...]
    ii = lax.broadcasted_iota(jnp.int32, (B, N), 1)
    bb = lax.broadcasted_iota(jnp.int32, (B, 1), 0) * N
    lpx = cx[:, 0:1]
    lpy = cy[:, 0:1]
    lpz = cz[:, 0:1]
    flat_ref[:, 0:1, :] = bb[:, :, None]
    ccx_ref[:, 0:1, :] = lpx[:, :, None]
    ccy_ref[:, 0:1, :] = lpy[:, :, None]
    ccz_ref[:, 0:1, :] = lpz[:, :, None]
    dist = None
    for t in range(1, NCEN):
        dx = cx - lpx
        dy = cy - lpy
        dz = cz - lpz
        d = dx * dx + dy * dy + dz * dz
        dist = d if dist is None else jnp.minimum(dist, d)
        m = jnp.max(dist, axis=1, keepdims=True)
        cand = jnp.where(dist == m, ii, jnp.int32(2 ** 30))
        nxt = jnp.min(cand, axis=1, keepdims=True)
        msk = ii == nxt
        lpx = jnp.sum(jnp.where(msk, cx, 0.0), axis=1, keepdims=True)
        lpy = jnp.sum(jnp.where(msk, cy, 0.0), axis=1, keepdims=True)
        lpz = jnp.sum(jnp.where(msk, cz, 0.0), axis=1, keepdims=True)
        flat_ref[:, t:t + 1, :] = (nxt + bb)[:, :, None]
        ccx_ref[:, t:t + 1, :] = lpx[:, :, None]
        ccy_ref[:, t:t + 1, :] = lpy[:, :, None]
        ccz_ref[:, t:t + 1, :] = lpz[:, :, None]


def _fps(cx, cy, cz):
    return pl.pallas_call(
        _fps_body,
        out_shape=[
            jax.ShapeDtypeStruct((B, NCEN, 1), jnp.int32),
            jax.ShapeDtypeStruct((B, NCEN, 1), jnp.float32),
            jax.ShapeDtypeStruct((B, NCEN, 1), jnp.float32),
            jax.ShapeDtypeStruct((B, NCEN, 1), jnp.float32),
        ],
    )(cx, cy, cz)


_NW = 32
_ROWS = B * NCEN
_RPW = _ROWS // _NW


def _sc_gather(table, idx):
    mesh = plsc.VectorSubcoreMesh(core_axis_name="c", subcore_axis_name="s")

    @functools.partial(
        pl.kernel,
        mesh=mesh,
        out_type=jax.ShapeDtypeStruct((_ROWS, D), jnp.float32),
        scratch_types=[
            pltpu.VMEM((_RPW,), jnp.int32),
            pltpu.VMEM((_RPW, D), jnp.float32),
            pltpu.SemaphoreType.DMA,
        ],
    )
    def gather_kernel(table_hbm, idx_hbm, out_hbm, idx_v, rows_v, sem):
        wid = lax.axis_index("s") * 2 + lax.axis_index("c")
        base = wid * _RPW
        pltpu.sync_copy(idx_hbm.at[pl.ds(base, _RPW)], idx_v)
        pltpu.async_copy(table_hbm.at[idx_v], rows_v, sem).wait()
        pltpu.sync_copy(rows_v, out_hbm.at[pl.ds(base, _RPW)])

    return gather_kernel(table, idx)


def _dot(a, b):
    return jax.lax.dot_general(a, b, (((1,), (0,)), ((), ())),
                               precision=_HIGH,
                               preferred_element_type=jnp.float32)


def _sigmoid(x):
    return 1.0 / (1.0 + jnp.exp(-x))


def _select_body(ce_ref, ccxc_ref, ccyc_ref, cczc_ref,
               ccxr_ref, ccyr_ref, cczr_ref,
               wg1_ref, bg1_ref, wg2_ref, bg2_ref,
               wc1_ref, bc1_ref, wc2_ref, bc2_ref,
               wd1_ref, bd1_ref, wd2_ref, bd2_ref,
               wp1_ref, bp1_ref, wp2_ref, bp2_ref,
               lng_ref, lnb_ref, out_ref, sel_ref, s_ref):
    ce = ce_ref[...]
    hg = jnp.maximum(_dot(ce, wg1_ref[...]) + bg1_ref[...], 0.0)
    pg = _sigmoid(_dot(hg, wg2_ref[...]) + bg2_ref[...])
    hc = jnp.maximum(_dot(ce, wc1_ref[...]) + bc1_ref[...], 0.0)
    pc = _sigmoid(_dot(hc, wc2_ref[...]) + bc2_ref[...])
    hd = jnp.maximum(_dot(ce, wd1_ref[...]) + bd1_ref[...], 0.0)
    pd = _sigmoid(_dot(hd, wd2_ref[...]) + bd2_ref[...])

    def topk_onehot(row, k, srow):
        w = row.shape[1]
        iot = lax.broadcasted_iota(jnp.int32, (1, w), 1)
        big = jnp.int32(2 ** 30)
        for j in range(k):
            m = jnp.max(row, axis=1, keepdims=True)
            idxv = jnp.min(jnp.where(row == m, iot, big), axis=1,
                           keepdims=True)
            oh = iot == idxv
            s_ref[srow + j:srow + j + 1, 0:w] = jnp.where(oh, 1.0, 0.0)
            row = jnp.where(oh, -jnp.inf, row)

    for b in range(B):
        s_ref[...] = jnp.zeros((40, NCEN), jnp.float32)
        z = cczc_ref[b]
        hr = _sigmoid((z - jnp.mean(z)) / 5.0)
        sg = 1.0 + hr * 0.95
        pg_b = pg[b * NCEN:(b + 1) * NCEN, :]
        score_g = jnp.mean(pg_b * sg, axis=1, keepdims=True)
        topk_onehot(jnp.reshape(score_g, (1, NCEN)), 16, 0)

        pc_b = pc[b * NCEN:b * NCEN + 128, :]
        score_c = jnp.mean(pc_b, axis=1, keepdims=True)
        topk_onehot(jnp.reshape(score_c, (1, 128)), 16, 16)

        pd_b = pd[b * NCEN:b * NCEN + 64, :]
        xi = ccxc_ref[b, 0:64, :]
        yi = ccyc_ref[b, 0:64, :]
        zi = cczc_ref[b, 0:64, :]
        xj = ccxr_ref[b:b + 1, 0:64]
        yj = ccyr_ref[b:b + 1, 0:64]
        zj = cczr_ref[b:b + 1, 0:64]
        dxx = xi - xj
        dyy = yi - yj
        dzz = zi - zj
        d2 = dxx * dxx + dyy * dyy + dzz * dzz
        dens = jnp.sum(jnp.where(d2 < 0.25, 1.0, 0.0), axis=1, keepdims=True)
        sd = 1.0 + dens / 64.0 * 0.95
        score_d = jnp.mean(pd_b * sd, axis=1, keepdims=True)
        topk_onehot(jnp.reshape(score_d, (1, 64)), 8, 32)

        ce_b = ce[b * NCEN:(b + 1) * NCEN, :]
        sel_ref[b * 40:(b + 1) * 40, :] = _dot(s_ref[...], ce_b)

    sel = sel_ref[...]
    h2 = jnp.maximum(_dot(sel, wp1_ref[...]) + bp1_ref[...], 0.0)
    o = _dot(h2, wp2_ref[...]) + bp2_ref[...]
    mu = jnp.mean(o, axis=1, keepdims=True)
    var = jnp.mean((o - mu) * (o - mu), axis=1, keepdims=True)
    res = (o - mu) / jnp.sqrt(var + 1e-5) * lng_ref[...] + lnb_ref[...]
    out_ref[...] = res.reshape(B, 40, D)


def _select(ce, ccxc, ccyc, cczc, ccxr, ccyr, cczr, *weights):
    return pl.pallas_call(
        _select_body,
        out_shape=jax.ShapeDtypeStruct((B, 40, D), jnp.float32),
        scratch_shapes=[pltpu.VMEM((B * 40, D), jnp.float32),
                        pltpu.VMEM((40, NCEN), jnp.float32)],
    )(ce, ccxc, ccyc, cczc, ccxr, ccyr, cczr, *weights)


def kernel(point_features, point_coords, Wg1, bg1, Wg2, bg2, Wc1, bc1, Wc2,
           bc2, Wd1, bd1, Wd2, bd2, Wp1, bp1, Wp2, bp2, ln_g, ln_b):
    cx = point_coords[:, :, 0]
    cy = point_coords[:, :, 1]
    cz = point_coords[:, :, 2]

    flat, ccx, ccy, ccz = _fps(cx, cy, cz)

    idx = flat.reshape(_ROWS)
    ce = _sc_gather(point_features.reshape(B * N, D), idx)

    out = _select(
        ce, ccx, ccy, ccz,
        ccx.reshape(B, NCEN), ccy.reshape(B, NCEN), ccz.reshape(B, NCEN),
        Wg1, bg1.reshape(1, -1), Wg2, bg2.reshape(1, -1),
        Wc1, bc1.reshape(1, -1), Wc2, bc2.reshape(1, -1),
        Wd1, bd1.reshape(1, -1), Wd2, bd2.reshape(1, -1),
        Wp1, bp1.reshape(1, -1), Wp2, bp2.reshape(1, -1),
        ln_g.reshape(1, -1), ln_b.reshape(1, -1),
    )
    return out
